# baseline (speedup 1.0000x reference)
import functools

import jax
import jax.numpy as jnp
from jax import lax
from jax.experimental import pallas as pl
from jax.experimental.pallas import tpu as pltpu
from jax.experimental.pallas import tpu_sc as plsc

_N = 10000
_E = 320000
_D = 128

_NC = 2
_NS = 16
_L = 16
_NW = _NC * _NS
_K = 112
_CH0 = 138
_CH1 = 42
_CHMAX = _CH0
_EP = _NS * (_CH0 + _CH1) * _K
_NSLOT = 3
_NIS = 4
_NPAD = 10112
_ROWS = _NPAD // _NS

_BA = 1024
_BC = 1024


def _attn_body(x_ref, w_ref, a_ref, h_ref, hbf_ref, s1_ref, s2_ref, cself_ref):
    hb = jnp.dot(x_ref[...], w_ref[...], preferred_element_type=jnp.float32)
    h_ref[...] = hb
    hbf_ref[...] = hb.astype(jnp.bfloat16)
    av = a_ref[0, :]
    s1 = jnp.dot(hb, av[:_D])
    s2 = jnp.dot(hb, av[_D:])
    e = s1 + s2
    s1_ref[...] = s1
    s2_ref[...] = s2
    cself_ref[...] = jnp.exp(jnp.maximum(e, 0.2 * e))


def _attn_call(x, W, a):
    grid = (-(-_N // _BA),)
    vec_spec = pl.BlockSpec((_BA,), lambda i: (i,))
    vec_shape = jax.ShapeDtypeStruct((_N,), jnp.float32)
    return pl.pallas_call(
        _attn_body,
        grid=grid,
        in_specs=[
            pl.BlockSpec((_BA, _D), lambda i: (i, 0)),
            pl.BlockSpec((_D, _D), lambda i: (0, 0)),
            pl.BlockSpec((1, 2 * _D), lambda i: (0, 0)),
        ],
        out_specs=[
            pl.BlockSpec((_BA, _D), lambda i: (i, 0)),
            pl.BlockSpec((_BA, _D), lambda i: (i, 0)),
            vec_spec, vec_spec, vec_spec,
        ],
        out_shape=[
            jax.ShapeDtypeStruct((_N, _D), jnp.float32),
            jax.ShapeDtypeStruct((_N, _D), jnp.bfloat16),
            vec_shape, vec_shape, vec_shape,
        ],
    )(x, W, a)


def _edge_body(h_hbm, s1_hbm, s2_hbm, src_hbm, dst_hbm,
               num_out, den_out,
               src_c, dst_c, c_c, s1g_v, s2g_v, rows_v, bounce_v,
               num_sh, den_sh, rsem, isem, ssem, wsem):
    cid = lax.axis_index("c")
    sid = lax.axis_index("s")
    wid = cid * _NS + sid
    r0 = sid * _ROWS

    zv = jnp.zeros((_L,), jnp.float32)

    def zrow(r, c):
        for q in range(_D // _L):
            rows_v[0, r, pl.ds(q * _L, _L)] = zv
        return c
    lax.fori_loop(0, _K, zrow, 0)
    for b in range(_ROWS // _L):
        bounce_v[pl.ds(b * _L, _L)] = zv
    bounce_v[pl.ds(_ROWS - _L, _L)] = zv
    off = 0
    while off < _ROWS:
        sz = min(_K, _ROWS - off)
        pltpu.sync_copy(rows_v.at[0, pl.ds(0, sz)],
                        num_sh.at[pl.ds(r0 + off, sz)])
        off += sz
    pltpu.sync_copy(bounce_v, den_sh.at[pl.ds(r0, _ROWS)])
    plsc.subcore_barrier()


    def launch_idx(j):
        ii = jnp.bitwise_and(j, _NIS - 1)
        pltpu.async_copy(src_hbm.at[wid, j], src_c.at[ii], isem.at[ii])
        pltpu.async_copy(dst_hbm.at[wid, j], dst_c.at[ii], isem.at[ii])

    def launch_gathers(j, k):
        ii = jnp.bitwise_and(j, _NIS - 1)
        pltpu.make_async_copy(src_hbm.at[wid, j], src_c.at[ii],
                              isem.at[ii]).wait()
        pltpu.make_async_copy(dst_hbm.at[wid, j], dst_c.at[ii],
                              isem.at[ii]).wait()
        _H = _K // 4
        for hq in range(4):
            pltpu.async_copy(h_hbm.at[dst_c.at[ii, pl.ds(hq * _H, _H)]],
                             rows_v.at[k, pl.ds(hq * _H, _H)], rsem.at[k])
        pltpu.async_copy(s1_hbm.at[src_c.at[ii]], s1g_v.at[k], ssem.at[k])
        pltpu.async_copy(s2_hbm.at[dst_c.at[ii]], s2g_v.at[k], ssem.at[k])

    def wait_s1s2(j, k):
        ii = jnp.bitwise_and(j, _NIS - 1)
        pltpu.make_async_copy(s1_hbm.at[src_c.at[ii]], s1g_v.at[k],
                              ssem.at[k]).wait()
        pltpu.make_async_copy(s2_hbm.at[dst_c.at[ii]], s2g_v.at[k],
                              ssem.at[k]).wait()

    def wait_rows(j, k):
        ii = jnp.bitwise_and(j, _NIS - 1)
        _H = _K // 4
        for hq in range(4):
            pltpu.make_async_copy(
                h_hbm.at[dst_c.at[ii, pl.ds(hq * _H, _H)]],
                rows_v.at[k, pl.ds(hq * _H, _H)], rsem.at[k]).wait()

    def start_scatter(j, k):
        ii = jnp.bitwise_and(j, _NIS - 1)
        pltpu.async_copy(rows_v.at[k], num_sh.at[src_c.at[ii]], wsem.at[k],
                         add=True)
        pltpu.async_copy(c_c.at[k], den_sh.at[src_c.at[ii]], wsem.at[k],
                         add=True)

    def wait_scatter(j, k):
        ii = jnp.bitwise_and(j, _NIS - 1)
        pltpu.make_async_copy(rows_v.at[k], num_sh.at[src_c.at[ii]],
                              wsem.at[k]).wait()
        pltpu.make_async_copy(c_c.at[k], den_sh.at[src_c.at[ii]],
                              wsem.at[k]).wait()

    def compute_c(j, k):
        ii = jnp.bitwise_and(j, _NIS - 1)
        for i in range(_K // _L):
            sv = src_c[ii, pl.ds(i * _L, _L)]
            dv = dst_c[ii, pl.ds(i * _L, _L)]
            e = s1g_v[k, pl.ds(i * _L, _L)] + s2g_v[k, pl.ds(i * _L, _L)]
            e = jnp.maximum(e, 0.2 * e)
            c = jnp.where(sv != dv, jnp.exp(e), 0.0)
            c_c[k, pl.ds(i * _L, _L)] = c

    def expand_scale(k):
        def scale_group(g, c2):
            cg = c_c[k, pl.ds(g * _L, _L)]
            for r16 in range(_L):
                r = g * _L + r16
                cb = jnp.broadcast_to(cg[r16], (_L,))
                for q in range(_D // _L):
                    rows_v[k, r, pl.ds(q * _L, _L)] = (
                        rows_v[k, r, pl.ds(q * _L, _L)] * cb)
            return c2
        lax.fori_loop(0, _K // _L, scale_group, 0)

    che = jnp.where(cid == 0, _CH0, _CH1)
    te = che // _NSLOT

    def sub(j, t, kk):
        cur, nxt = kk, (kk + 1) % _NSLOT
        if kk == 2:
            wait_scatter(j - 2, nxt)
        else:
            @pl.when(t > 0)
            def _():
                wait_scatter(j - 2, nxt)
        if kk == 0:
            launch_idx(j + 2)
        else:
            @pl.when(t < te - 1)
            def _():
                launch_idx(j + 2)
        if kk == 2:
            @pl.when(t < te - 1)
            def _():
                launch_gathers(j + 1, nxt)
        else:
            launch_gathers(j + 1, nxt)
        wait_s1s2(j, cur)
        compute_c(j, cur)
        wait_rows(j, cur)
        expand_scale(cur)
        start_scatter(j, cur)

    launch_idx(0)
    launch_idx(1)
    launch_gathers(0, 0)

    def body(t, carry):
        sub(3 * t + 0, t, 0)
        sub(3 * t + 1, t, 1)
        sub(3 * t + 2, t, 2)
        return carry
    lax.fori_loop(0, te, body, 0)

    wait_scatter(che - 2, 1)
    wait_scatter(che - 1, 2)

    plsc.subcore_barrier()

    pltpu.sync_copy(num_sh.at[pl.ds(r0, _ROWS)], num_out.at[cid, pl.ds(r0, _ROWS)])
    pltpu.sync_copy(den_sh.at[pl.ds(r0, _ROWS)], bounce_v)
    pltpu.sync_copy(bounce_v,
                    den_out.at[pl.ds(cid * _NPAD + r0, _ROWS)])


def _edge_call(h, s1, s2, srcp, dstp):
    mesh = plsc.VectorSubcoreMesh(
        core_axis_name="c", subcore_axis_name="s",
        num_cores=_NC, num_subcores=_NS)
    return pl.kernel(
        _edge_body,
        out_type=(
            jax.ShapeDtypeStruct((_NC, _NPAD, _D), jnp.float32),
            jax.ShapeDtypeStruct((_NC * _NPAD,), jnp.float32),
        ),
        mesh=mesh,
        scratch_types=[
            pltpu.VMEM((_NIS, _K), jnp.int32),
            pltpu.VMEM((_NIS, _K), jnp.int32),
            pltpu.VMEM((_NSLOT, _K), jnp.float32),
            pltpu.VMEM((_NSLOT, _K), jnp.float32),
            pltpu.VMEM((_NSLOT, _K), jnp.float32),
            pltpu.VMEM((_NSLOT, _K, _D), jnp.float32),
            pltpu.VMEM((_ROWS,), jnp.float32),
            pltpu.VMEM_SHARED((_NPAD, _D), jnp.float32),
            pltpu.VMEM_SHARED((_NPAD,), jnp.float32),
            pltpu.SemaphoreType.DMA((_NSLOT,)),
            pltpu.SemaphoreType.DMA((_NIS,)),
            pltpu.SemaphoreType.DMA((_NSLOT,)),
            pltpu.SemaphoreType.DMA((_NSLOT,)),
        ],
        compiler_params=pltpu.CompilerParams(needs_layout_passes=False),
    )(h, s1, s2, srcp, dstp)


def _combine_body(num_ref, den_ref, h_ref, cself_ref, out_ref):
    cself = cself_ref[...]
    numsum = num_ref[0] + num_ref[1] + cself[:, None] * h_ref[...]
    densum = den_ref[0] + den_ref[1] + cself
    out_ref[...] = numsum / densum[:, None]


def _combine_call(num, den, h, cself):
    grid = (-(-_N // _BC),)
    return pl.pallas_call(
        _combine_body,
        grid=grid,
        in_specs=[
            pl.BlockSpec((_NC, _BC, _D), lambda i: (0, i, 0)),
            pl.BlockSpec((_NC, _BC), lambda i: (0, i)),
            pl.BlockSpec((_BC, _D), lambda i: (i, 0)),
            pl.BlockSpec((_BC,), lambda i: (i,)),
        ],
        out_specs=pl.BlockSpec((_BC, _D), lambda i: (i, 0)),
        out_shape=jax.ShapeDtypeStruct((_N, _D), jnp.float32),
    )(num, den, h, cself)


def kernel(x, edge_index, W, a):
    src = edge_index[0].astype(jnp.int32)
    dst = edge_index[1].astype(jnp.int32)
    pad = _EP - _E
    def _split(v):
        vp = jnp.concatenate([v, jnp.zeros((pad,), jnp.int32)])
        n0 = _NS * _CH0 * _K
        e0 = vp[:n0].reshape(_NS, _CH0, _K)
        e1 = vp[n0:].reshape(_NS, _CH1, _K)
        e1 = jnp.pad(e1, ((0, 0), (0, _CH0 - _CH1), (0, 0)))
        return jnp.concatenate([e0, e1], axis=0)

    srcp = _split(src)
    dstp = _split(dst)

    h, hbf, s1, s2, cself = _attn_call(x, W, a)

    num, den = _edge_call(h, s1, s2, srcp, dstp)
    den = den.reshape(_NC, _NPAD)

    return _combine_call(num[:, :_N], den[:, :_N], h, cself)


# speedup vs baseline: 1.0069x; 1.0069x over previous
import functools

import jax
import jax.numpy as jnp
from jax import lax
from jax.experimental import pallas as pl
from jax.experimental.pallas import tpu as pltpu
from jax.experimental.pallas import tpu_sc as plsc

_N = 10000
_E = 320000
_D = 128

_NC = 2
_NS = 16
_L = 16
_NW = _NC * _NS
_K = 112
_CH0 = 147
_CH1 = 33
_CHMAX = _CH0
_EP = _NS * (_CH0 + _CH1) * _K
_NSLOT = 3
_NIS = 4
_NPAD = 10112
_ROWS = _NPAD // _NS

_BA = 1024
_BC = 1024


def _attn_body(x_ref, w_ref, a_ref, h_ref, hbf_ref, s1_ref, s2_ref, cself_ref):
    hb = jnp.dot(x_ref[...], w_ref[...], preferred_element_type=jnp.float32)
    h_ref[...] = hb
    hbf_ref[...] = hb.astype(jnp.bfloat16)
    av = a_ref[0, :]
    s1 = jnp.dot(hb, av[:_D])
    s2 = jnp.dot(hb, av[_D:])
    e = s1 + s2
    s1_ref[...] = s1
    s2_ref[...] = s2
    cself_ref[...] = jnp.exp(jnp.maximum(e, 0.2 * e))


def _attn_call(x, W, a):
    grid = (-(-_N // _BA),)
    vec_spec = pl.BlockSpec((_BA,), lambda i: (i,))
    vec_shape = jax.ShapeDtypeStruct((_N,), jnp.float32)
    return pl.pallas_call(
        _attn_body,
        grid=grid,
        in_specs=[
            pl.BlockSpec((_BA, _D), lambda i: (i, 0)),
            pl.BlockSpec((_D, _D), lambda i: (0, 0)),
            pl.BlockSpec((1, 2 * _D), lambda i: (0, 0)),
        ],
        out_specs=[
            pl.BlockSpec((_BA, _D), lambda i: (i, 0)),
            pl.BlockSpec((_BA, _D), lambda i: (i, 0)),
            vec_spec, vec_spec, vec_spec,
        ],
        out_shape=[
            jax.ShapeDtypeStruct((_N, _D), jnp.float32),
            jax.ShapeDtypeStruct((_N, _D), jnp.bfloat16),
            vec_shape, vec_shape, vec_shape,
        ],
    )(x, W, a)


def _edge_body(h_hbm, s1_hbm, s2_hbm, src_hbm, dst_hbm,
               num_out, den_out,
               src_c, dst_c, c_c, s1g_v, s2g_v, rows_v, bounce_v,
               num_sh, den_sh, rsem, isem, ssem, wsem):
    cid = lax.axis_index("c")
    sid = lax.axis_index("s")
    wid = cid * _NS + sid
    r0 = sid * _ROWS

    zv = jnp.zeros((_L,), jnp.float32)

    def zrow(r, c):
        for q in range(_D // _L):
            rows_v[0, r, pl.ds(q * _L, _L)] = zv
        return c
    lax.fori_loop(0, _K, zrow, 0)
    for b in range(_ROWS // _L):
        bounce_v[pl.ds(b * _L, _L)] = zv
    bounce_v[pl.ds(_ROWS - _L, _L)] = zv
    off = 0
    while off < _ROWS:
        sz = min(_K, _ROWS - off)
        pltpu.sync_copy(rows_v.at[0, pl.ds(0, sz)],
                        num_sh.at[pl.ds(r0 + off, sz)])
        off += sz
    pltpu.sync_copy(bounce_v, den_sh.at[pl.ds(r0, _ROWS)])
    plsc.subcore_barrier()


    def launch_idx(j):
        ii = jnp.bitwise_and(j, _NIS - 1)
        pltpu.async_copy(src_hbm.at[wid, j], src_c.at[ii], isem.at[ii])
        pltpu.async_copy(dst_hbm.at[wid, j], dst_c.at[ii], isem.at[ii])

    def launch_gathers(j, k):
        ii = jnp.bitwise_and(j, _NIS - 1)
        pltpu.make_async_copy(src_hbm.at[wid, j], src_c.at[ii],
                              isem.at[ii]).wait()
        pltpu.make_async_copy(dst_hbm.at[wid, j], dst_c.at[ii],
                              isem.at[ii]).wait()
        _H = _K // 4
        for hq in range(4):
            pltpu.async_copy(h_hbm.at[dst_c.at[ii, pl.ds(hq * _H, _H)]],
                             rows_v.at[k, pl.ds(hq * _H, _H)], rsem.at[k])
        pltpu.async_copy(s1_hbm.at[src_c.at[ii]], s1g_v.at[k], ssem.at[k])
        pltpu.async_copy(s2_hbm.at[dst_c.at[ii]], s2g_v.at[k], ssem.at[k])

    def wait_s1s2(j, k):
        ii = jnp.bitwise_and(j, _NIS - 1)
        pltpu.make_async_copy(s1_hbm.at[src_c.at[ii]], s1g_v.at[k],
                              ssem.at[k]).wait()
        pltpu.make_async_copy(s2_hbm.at[dst_c.at[ii]], s2g_v.at[k],
                              ssem.at[k]).wait()

    def wait_rows(j, k):
        ii = jnp.bitwise_and(j, _NIS - 1)
        _H = _K // 4
        for hq in range(4):
            pltpu.make_async_copy(
                h_hbm.at[dst_c.at[ii, pl.ds(hq * _H, _H)]],
                rows_v.at[k, pl.ds(hq * _H, _H)], rsem.at[k]).wait()

    def start_scatter(j, k):
        ii = jnp.bitwise_and(j, _NIS - 1)
        pltpu.async_copy(rows_v.at[k], num_sh.at[src_c.at[ii]], wsem.at[k],
                         add=True)
        pltpu.async_copy(c_c.at[k], den_sh.at[src_c.at[ii]], wsem.at[k],
                         add=True)

    def wait_scatter(j, k):
        ii = jnp.bitwise_and(j, _NIS - 1)
        pltpu.make_async_copy(rows_v.at[k], num_sh.at[src_c.at[ii]],
                              wsem.at[k]).wait()
        pltpu.make_async_copy(c_c.at[k], den_sh.at[src_c.at[ii]],
                              wsem.at[k]).wait()

    def compute_c(j, k):
        ii = jnp.bitwise_and(j, _NIS - 1)
        for i in range(_K // _L):
            sv = src_c[ii, pl.ds(i * _L, _L)]
            dv = dst_c[ii, pl.ds(i * _L, _L)]
            e = s1g_v[k, pl.ds(i * _L, _L)] + s2g_v[k, pl.ds(i * _L, _L)]
            e = jnp.maximum(e, 0.2 * e)
            c = jnp.where(sv != dv, jnp.exp(e), 0.0)
            c_c[k, pl.ds(i * _L, _L)] = c

    def expand_scale(k):
        def scale_group(g, c2):
            cg = c_c[k, pl.ds(g * _L, _L)]
            for r16 in range(_L):
                r = g * _L + r16
                cb = jnp.broadcast_to(cg[r16], (_L,))
                for q in range(_D // _L):
                    rows_v[k, r, pl.ds(q * _L, _L)] = (
                        rows_v[k, r, pl.ds(q * _L, _L)] * cb)
            return c2
        lax.fori_loop(0, _K // _L, scale_group, 0)

    che = jnp.where(cid == 0, _CH0, _CH1)
    te = che // _NSLOT

    def sub(j, t, kk):
        cur, nxt = kk, (kk + 1) % _NSLOT
        if kk == 2:
            wait_scatter(j - 2, nxt)
        else:
            @pl.when(t > 0)
            def _():
                wait_scatter(j - 2, nxt)
        if kk == 0:
            launch_idx(j + 2)
        else:
            @pl.when(t < te - 1)
            def _():
                launch_idx(j + 2)
        if kk == 2:
            @pl.when(t < te - 1)
            def _():
                launch_gathers(j + 1, nxt)
        else:
            launch_gathers(j + 1, nxt)
        wait_s1s2(j, cur)
        compute_c(j, cur)
        wait_rows(j, cur)
        expand_scale(cur)
        start_scatter(j, cur)

    launch_idx(0)
    launch_idx(1)
    launch_gathers(0, 0)

    def body(t, carry):
        sub(3 * t + 0, t, 0)
        sub(3 * t + 1, t, 1)
        sub(3 * t + 2, t, 2)
        return carry
    lax.fori_loop(0, te, body, 0)

    wait_scatter(che - 2, 1)
    wait_scatter(che - 1, 2)

    plsc.subcore_barrier()

    pltpu.sync_copy(num_sh.at[pl.ds(r0, _ROWS)], num_out.at[cid, pl.ds(r0, _ROWS)])
    pltpu.sync_copy(den_sh.at[pl.ds(r0, _ROWS)], bounce_v)
    pltpu.sync_copy(bounce_v,
                    den_out.at[pl.ds(cid * _NPAD + r0, _ROWS)])


def _edge_call(h, s1, s2, srcp, dstp):
    mesh = plsc.VectorSubcoreMesh(
        core_axis_name="c", subcore_axis_name="s",
        num_cores=_NC, num_subcores=_NS)
    return pl.kernel(
        _edge_body,
        out_type=(
            jax.ShapeDtypeStruct((_NC, _NPAD, _D), jnp.float32),
            jax.ShapeDtypeStruct((_NC * _NPAD,), jnp.float32),
        ),
        mesh=mesh,
        scratch_types=[
            pltpu.VMEM((_NIS, _K), jnp.int32),
            pltpu.VMEM((_NIS, _K), jnp.int32),
            pltpu.VMEM((_NSLOT, _K), jnp.float32),
            pltpu.VMEM((_NSLOT, _K), jnp.float32),
            pltpu.VMEM((_NSLOT, _K), jnp.float32),
            pltpu.VMEM((_NSLOT, _K, _D), jnp.float32),
            pltpu.VMEM((_ROWS,), jnp.float32),
            pltpu.VMEM_SHARED((_NPAD, _D), jnp.float32),
            pltpu.VMEM_SHARED((_NPAD,), jnp.float32),
            pltpu.SemaphoreType.DMA((_NSLOT,)),
            pltpu.SemaphoreType.DMA((_NIS,)),
            pltpu.SemaphoreType.DMA((_NSLOT,)),
            pltpu.SemaphoreType.DMA((_NSLOT,)),
        ],
        compiler_params=pltpu.CompilerParams(needs_layout_passes=False),
    )(h, s1, s2, srcp, dstp)


def _combine_body(num_ref, den_ref, h_ref, cself_ref, out_ref):
    cself = cself_ref[...]
    numsum = num_ref[0] + num_ref[1] + cself[:, None] * h_ref[...]
    densum = den_ref[0] + den_ref[1] + cself
    out_ref[...] = numsum / densum[:, None]


def _combine_call(num, den, h, cself):
    grid = (-(-_N // _BC),)
    return pl.pallas_call(
        _combine_body,
        grid=grid,
        in_specs=[
            pl.BlockSpec((_NC, _BC, _D), lambda i: (0, i, 0)),
            pl.BlockSpec((_NC, _BC), lambda i: (0, i)),
            pl.BlockSpec((_BC, _D), lambda i: (i, 0)),
            pl.BlockSpec((_BC,), lambda i: (i,)),
        ],
        out_specs=pl.BlockSpec((_BC, _D), lambda i: (i, 0)),
        out_shape=jax.ShapeDtypeStruct((_N, _D), jnp.float32),
    )(num, den, h, cself)


def kernel(x, edge_index, W, a):
    src = edge_index[0].astype(jnp.int32)
    dst = edge_index[1].astype(jnp.int32)
    pad = _EP - _E
    def _split(v):
        vp = jnp.concatenate([v, jnp.zeros((pad,), jnp.int32)])
        n0 = _NS * _CH0 * _K
        e0 = vp[:n0].reshape(_NS, _CH0, _K)
        e1 = vp[n0:].reshape(_NS, _CH1, _K)
        e1 = jnp.pad(e1, ((0, 0), (0, _CH0 - _CH1), (0, 0)))
        return jnp.concatenate([e0, e1], axis=0)

    srcp = _split(src)
    dstp = _split(dst)

    h, hbf, s1, s2, cself = _attn_call(x, W, a)

    num, den = _edge_call(h, s1, s2, srcp, dstp)
    den = den.reshape(_NC, _NPAD)

    return _combine_call(num[:, :_N], den[:, :_N], h, cself)


# speedup vs baseline: 1.0094x; 1.0025x over previous
import functools

import jax
import jax.numpy as jnp
from jax import lax
from jax.experimental import pallas as pl
from jax.experimental.pallas import tpu as pltpu
from jax.experimental.pallas import tpu_sc as plsc

_N = 10000
_E = 320000
_D = 128

_NC = 2
_NS = 16
_L = 16
_NW = _NC * _NS
_K = 112
_CH0 = 147
_CH1 = 33
_CHMAX = _CH0
_EP = _NS * (_CH0 + _CH1) * _K
_NSLOT = 3
_NIS = 4
_NPAD = 10112
_ROWS = _NPAD // _NS

_BA = 1024
_BC = 1024


def _attn_body(x_ref, w_ref, a_ref, h_ref, s1_ref, s2_ref, cself_ref):
    hb = jnp.dot(x_ref[...], w_ref[...], preferred_element_type=jnp.float32)
    h_ref[...] = hb
    av = a_ref[0, :]
    s1 = jnp.dot(hb, av[:_D])
    s2 = jnp.dot(hb, av[_D:])
    e = s1 + s2
    s1_ref[...] = s1
    s2_ref[...] = s2
    cself_ref[...] = jnp.exp(jnp.maximum(e, 0.2 * e))


def _attn_call(x, W, a):
    grid = (-(-_N // _BA),)
    vec_spec = pl.BlockSpec((_BA,), lambda i: (i,))
    vec_shape = jax.ShapeDtypeStruct((_N,), jnp.float32)
    return pl.pallas_call(
        _attn_body,
        grid=grid,
        in_specs=[
            pl.BlockSpec((_BA, _D), lambda i: (i, 0)),
            pl.BlockSpec((_D, _D), lambda i: (0, 0)),
            pl.BlockSpec((1, 2 * _D), lambda i: (0, 0)),
        ],
        out_specs=[
            pl.BlockSpec((_BA, _D), lambda i: (i, 0)),
            vec_spec, vec_spec, vec_spec,
        ],
        out_shape=[
            jax.ShapeDtypeStruct((_N, _D), jnp.float32),
            vec_shape, vec_shape, vec_shape,
        ],
    )(x, W, a)


def _edge_body(h_hbm, s1_hbm, s2_hbm, src_hbm, dst_hbm,
               num_out, den_out,
               src_c, dst_c, c_c, s1g_v, s2g_v, rows_v, bounce_v,
               num_sh, den_sh, rsem, isem, ssem, wsem):
    cid = lax.axis_index("c")
    sid = lax.axis_index("s")
    wid = cid * _NS + sid
    r0 = sid * _ROWS

    zv = jnp.zeros((_L,), jnp.float32)

    def zrow(r, c):
        for q in range(_D // _L):
            rows_v[0, r, pl.ds(q * _L, _L)] = zv
        return c
    lax.fori_loop(0, _K, zrow, 0)
    for b in range(_ROWS // _L):
        bounce_v[pl.ds(b * _L, _L)] = zv
    bounce_v[pl.ds(_ROWS - _L, _L)] = zv
    off = 0
    while off < _ROWS:
        sz = min(_K, _ROWS - off)
        pltpu.sync_copy(rows_v.at[0, pl.ds(0, sz)],
                        num_sh.at[pl.ds(r0 + off, sz)])
        off += sz
    pltpu.sync_copy(bounce_v, den_sh.at[pl.ds(r0, _ROWS)])
    plsc.subcore_barrier()


    def launch_idx(j):
        ii = jnp.bitwise_and(j, _NIS - 1)
        pltpu.async_copy(src_hbm.at[wid, j], src_c.at[ii], isem.at[ii])
        pltpu.async_copy(dst_hbm.at[wid, j], dst_c.at[ii], isem.at[ii])

    def launch_gathers(j, k):
        ii = jnp.bitwise_and(j, _NIS - 1)
        pltpu.make_async_copy(src_hbm.at[wid, j], src_c.at[ii],
                              isem.at[ii]).wait()
        pltpu.make_async_copy(dst_hbm.at[wid, j], dst_c.at[ii],
                              isem.at[ii]).wait()
        _H = _K // 4
        for hq in range(4):
            pltpu.async_copy(h_hbm.at[dst_c.at[ii, pl.ds(hq * _H, _H)]],
                             rows_v.at[k, pl.ds(hq * _H, _H)], rsem.at[k])
        pltpu.async_copy(s1_hbm.at[src_c.at[ii]], s1g_v.at[k], ssem.at[k])
        pltpu.async_copy(s2_hbm.at[dst_c.at[ii]], s2g_v.at[k], ssem.at[k])

    def wait_s1s2(j, k):
        ii = jnp.bitwise_and(j, _NIS - 1)
        pltpu.make_async_copy(s1_hbm.at[src_c.at[ii]], s1g_v.at[k],
                              ssem.at[k]).wait()
        pltpu.make_async_copy(s2_hbm.at[dst_c.at[ii]], s2g_v.at[k],
                              ssem.at[k]).wait()

    def wait_rows(j, k):
        ii = jnp.bitwise_and(j, _NIS - 1)
        _H = _K // 4
        for hq in range(4):
            pltpu.make_async_copy(
                h_hbm.at[dst_c.at[ii, pl.ds(hq * _H, _H)]],
                rows_v.at[k, pl.ds(hq * _H, _H)], rsem.at[k]).wait()

    def start_scatter(j, k):
        ii = jnp.bitwise_and(j, _NIS - 1)
        pltpu.async_copy(rows_v.at[k], num_sh.at[src_c.at[ii]], wsem.at[k],
                         add=True)
        pltpu.async_copy(c_c.at[k], den_sh.at[src_c.at[ii]], wsem.at[k],
                         add=True)

    def wait_scatter(j, k):
        ii = jnp.bitwise_and(j, _NIS - 1)
        pltpu.make_async_copy(rows_v.at[k], num_sh.at[src_c.at[ii]],
                              wsem.at[k]).wait()
        pltpu.make_async_copy(c_c.at[k], den_sh.at[src_c.at[ii]],
                              wsem.at[k]).wait()

    def compute_c(j, k):
        ii = jnp.bitwise_and(j, _NIS - 1)
        for i in range(_K // _L):
            sv = src_c[ii, pl.ds(i * _L, _L)]
            dv = dst_c[ii, pl.ds(i * _L, _L)]
            e = s1g_v[k, pl.ds(i * _L, _L)] + s2g_v[k, pl.ds(i * _L, _L)]
            e = jnp.maximum(e, 0.2 * e)
            c = jnp.where(sv != dv, jnp.exp(e), 0.0)
            c_c[k, pl.ds(i * _L, _L)] = c

    def expand_scale(k):
        def scale_group(g, c2):
            cg = c_c[k, pl.ds(g * _L, _L)]
            for r16 in range(_L):
                r = g * _L + r16
                cb = jnp.broadcast_to(cg[r16], (_L,))
                for q in range(_D // _L):
                    rows_v[k, r, pl.ds(q * _L, _L)] = (
                        rows_v[k, r, pl.ds(q * _L, _L)] * cb)
            return c2
        lax.fori_loop(0, _K // _L, scale_group, 0)

    che = jnp.where(cid == 0, _CH0, _CH1)
    te = che // _NSLOT

    def sub(j, t, kk):
        cur, nxt = kk, (kk + 1) % _NSLOT
        if kk == 2:
            wait_scatter(j - 2, nxt)
        else:
            @pl.when(t > 0)
            def _():
                wait_scatter(j - 2, nxt)
        if kk == 0:
            launch_idx(j + 2)
        else:
            @pl.when(t < te - 1)
            def _():
                launch_idx(j + 2)
        if kk == 2:
            @pl.when(t < te - 1)
            def _():
                launch_gathers(j + 1, nxt)
        else:
            launch_gathers(j + 1, nxt)
        wait_s1s2(j, cur)
        compute_c(j, cur)
        wait_rows(j, cur)
        expand_scale(cur)
        start_scatter(j, cur)

    launch_idx(0)
    launch_idx(1)
    launch_gathers(0, 0)

    def body(t, carry):
        sub(3 * t + 0, t, 0)
        sub(3 * t + 1, t, 1)
        sub(3 * t + 2, t, 2)
        return carry
    lax.fori_loop(0, te, body, 0)

    wait_scatter(che - 2, 1)
    wait_scatter(che - 1, 2)

    plsc.subcore_barrier()

    pltpu.sync_copy(num_sh.at[pl.ds(r0, _ROWS)], num_out.at[cid, pl.ds(r0, _ROWS)])
    pltpu.sync_copy(den_sh.at[pl.ds(r0, _ROWS)], bounce_v)
    pltpu.sync_copy(bounce_v,
                    den_out.at[pl.ds(cid * _NPAD + r0, _ROWS)])


def _edge_call(h, s1, s2, srcp, dstp):
    mesh = plsc.VectorSubcoreMesh(
        core_axis_name="c", subcore_axis_name="s",
        num_cores=_NC, num_subcores=_NS)
    return pl.kernel(
        _edge_body,
        out_type=(
            jax.ShapeDtypeStruct((_NC, _NPAD, _D), jnp.float32),
            jax.ShapeDtypeStruct((_NC * _NPAD,), jnp.float32),
        ),
        mesh=mesh,
        scratch_types=[
            pltpu.VMEM((_NIS, _K), jnp.int32),
            pltpu.VMEM((_NIS, _K), jnp.int32),
            pltpu.VMEM((_NSLOT, _K), jnp.float32),
            pltpu.VMEM((_NSLOT, _K), jnp.float32),
            pltpu.VMEM((_NSLOT, _K), jnp.float32),
            pltpu.VMEM((_NSLOT, _K, _D), jnp.float32),
            pltpu.VMEM((_ROWS,), jnp.float32),
            pltpu.VMEM_SHARED((_NPAD, _D), jnp.float32),
            pltpu.VMEM_SHARED((_NPAD,), jnp.float32),
            pltpu.SemaphoreType.DMA((_NSLOT,)),
            pltpu.SemaphoreType.DMA((_NIS,)),
            pltpu.SemaphoreType.DMA((_NSLOT,)),
            pltpu.SemaphoreType.DMA((_NSLOT,)),
        ],
        compiler_params=pltpu.CompilerParams(needs_layout_passes=False),
    )(h, s1, s2, srcp, dstp)


def _combine_body(num_ref, den_ref, h_ref, cself_ref, out_ref):
    cself = cself_ref[...]
    numsum = num_ref[0] + num_ref[1] + cself[:, None] * h_ref[...]
    densum = den_ref[0] + den_ref[1] + cself
    out_ref[...] = numsum / densum[:, None]


def _combine_call(num, den, h, cself):
    grid = (-(-_N // _BC),)
    return pl.pallas_call(
        _combine_body,
        grid=grid,
        in_specs=[
            pl.BlockSpec((_NC, _BC, _D), lambda i: (0, i, 0)),
            pl.BlockSpec((_NC, _BC), lambda i: (0, i)),
            pl.BlockSpec((_BC, _D), lambda i: (i, 0)),
            pl.BlockSpec((_BC,), lambda i: (i,)),
        ],
        out_specs=pl.BlockSpec((_BC, _D), lambda i: (i, 0)),
        out_shape=jax.ShapeDtypeStruct((_N, _D), jnp.float32),
    )(num, den, h, cself)


def kernel(x, edge_index, W, a):
    src = edge_index[0].astype(jnp.int32)
    dst = edge_index[1].astype(jnp.int32)
    pad = _EP - _E
    def _split(v):
        vp = jnp.concatenate([v, jnp.zeros((pad,), jnp.int32)])
        n0 = _NS * _CH0 * _K
        e0 = vp[:n0].reshape(_NS, _CH0, _K)
        e1 = vp[n0:].reshape(_NS, _CH1, _K)
        e1 = jnp.pad(e1, ((0, 0), (0, _CH0 - _CH1), (0, 0)))
        return jnp.concatenate([e0, e1], axis=0)

    srcp = _split(src)
    dstp = _split(dst)

    h, s1, s2, cself = _attn_call(x, W, a)

    num, den = _edge_call(h, s1, s2, srcp, dstp)
    den = den.reshape(_NC, _NPAD)

    return _combine_call(num[:, :_N], den[:, :_N], h, cself)
